# baseline (speedup 1.0000x reference)
import functools

import jax
import jax.numpy as jnp
from jax import lax
from jax.experimental import pallas as pl
from jax.experimental.pallas import tpu as pltpu

VOCAB = 1000
BATCH = 4096
SEQ = 20
B_TOTAL = BATCH * SEQ
R = 512
NB = B_TOTAL // R


def _onehot_body(idx_ref, tab_ref, out_ref):
    idx_blk = idx_ref[0]
    iot = lax.broadcasted_iota(jnp.int32, (R, VOCAB), 1)
    oh = (iot == idx_blk).astype(jnp.bfloat16)
    out_ref[...] = jnp.dot(oh, tab_ref[...],
                           preferred_element_type=jnp.float32)


@jax.jit
def _bigram_logits(table, idx_flat):
    idx3 = idx_flat.reshape(NB, R, 1)
    table_bf = table.astype(jnp.bfloat16)
    return pl.pallas_call(
        _onehot_body,
        grid=(NB,),
        in_specs=[
            pl.BlockSpec((1, R, 1), lambda i: (i, 0, 0)),
            pl.BlockSpec((VOCAB, VOCAB), lambda i: (0, 0)),
        ],
        out_specs=pl.BlockSpec((R, VOCAB), lambda i: (i, 0)),
        out_shape=jax.ShapeDtypeStruct((B_TOTAL, VOCAB), jnp.float32),
        compiler_params=pltpu.CompilerParams(
            dimension_semantics=("arbitrary",)),
    )(idx3, table_bf)


def kernel(inputs, table):
    idx_flat = inputs.reshape(-1).astype(jnp.int32)
    out = _bigram_logits(table, idx_flat)
    return out.reshape(BATCH, SEQ, VOCAB)


# speedup vs baseline: 1.1557x; 1.1557x over previous
import functools

import jax
import jax.numpy as jnp
from jax import lax
from jax.experimental import pallas as pl
from jax.experimental.pallas import tpu as pltpu
from jax.experimental.pallas import tpu_sc as plsc

VOCAB = 1000
BATCH = 4096
SEQ = 20
B_TOTAL = BATCH * SEQ
NUM_CORES = 2
NUM_SUBCORES = 16
NW = NUM_CORES * NUM_SUBCORES
B_PER_W = B_TOTAL // NW
NBUF = 4
K = 16
NCHUNK = B_PER_W // K
AHEAD = NBUF - 2
STAGE_TILES = 8
STAGE_ROWS = VOCAB // STAGE_TILES


def _gather_kernel(table_hbm, idx_hbm, out_hbm, shared, idx_v,
                   rows0, rows1, rows2, rows3,
                   gsem0, gsem1, gsem2, gsem3,
                   ssem0, ssem1, ssem2, ssem3):
    sid = lax.axis_index("s")
    wid = sid * NUM_CORES + lax.axis_index("c")
    base = wid * B_PER_W

    rows = (rows0, rows1, rows2, rows3)
    gsems = (gsem0, gsem1, gsem2, gsem3)
    ssems = (ssem0, ssem1, ssem2, ssem3)

    @pl.when(sid < STAGE_TILES)
    def _():
        pltpu.sync_copy(
            table_hbm.at[pl.ds(sid * STAGE_ROWS, STAGE_ROWS)],
            shared.at[pl.ds(sid * STAGE_ROWS, STAGE_ROWS)])

    pltpu.sync_copy(idx_hbm.at[pl.ds(base, B_PER_W)], idx_v)
    plsc.subcore_barrier()

    def gather_start(g, p):
        pltpu.async_copy(
            shared.at[idx_v.at[pl.ds(g * K, K)]], rows[p], gsems[p])

    def gather_wait(p):
        pltpu.make_async_copy(
            shared.at[pl.ds(0, K)], rows[p], gsems[p]).wait()

    def scatter_start(g, p):
        pltpu.async_copy(
            rows[p], out_hbm.at[pl.ds(base + g * K, K)], ssems[p])

    def scatter_wait(p):
        pltpu.make_async_copy(
            rows[p], out_hbm.at[pl.ds(base, K)], ssems[p]).wait()

    for p in range(AHEAD):
        gather_start(p, p)

    def round_body(m, carry):
        for p in range(NBUF):
            g = m * NBUF + p
            gather_wait(p)
            scatter_start(g, p)
            pq = (p + NBUF - AHEAD) % NBUF

            @pl.when(g >= AHEAD)
            def _():
                scatter_wait(pq)

            @pl.when(g + NBUF - AHEAD < NCHUNK)
            def _():
                gather_start(g + NBUF - AHEAD, pq)
        return carry

    lax.fori_loop(0, NCHUNK // NBUF, round_body, 0)

    for g in range(NCHUNK - AHEAD, NCHUNK):
        scatter_wait(g % NBUF)


@jax.jit
def _bigram_logits(table, idx_flat):
    mesh = plsc.VectorSubcoreMesh(core_axis_name="c", subcore_axis_name="s")
    run = functools.partial(
        pl.kernel,
        out_type=jax.ShapeDtypeStruct((B_TOTAL, VOCAB), jnp.float32),
        mesh=mesh,
        scratch_types=[
            pltpu.VMEM_SHARED((VOCAB, VOCAB), jnp.float32),
            pltpu.VMEM((B_PER_W,), jnp.int32),
            pltpu.VMEM((K, VOCAB), jnp.float32),
            pltpu.VMEM((K, VOCAB), jnp.float32),
            pltpu.VMEM((K, VOCAB), jnp.float32),
            pltpu.VMEM((K, VOCAB), jnp.float32),
            pltpu.SemaphoreType.DMA,
            pltpu.SemaphoreType.DMA,
            pltpu.SemaphoreType.DMA,
            pltpu.SemaphoreType.DMA,
            pltpu.SemaphoreType.DMA,
            pltpu.SemaphoreType.DMA,
            pltpu.SemaphoreType.DMA,
            pltpu.SemaphoreType.DMA,
        ],
        compiler_params=pltpu.CompilerParams(use_tc_tiling_on_sc=False),
    )(_gather_kernel)
    return run(table, idx_flat)


def kernel(inputs, table):
    idx_flat = inputs.reshape(-1).astype(jnp.int32)
    out = _bigram_logits(table, idx_flat)
    return out.reshape(BATCH, SEQ, VOCAB)


# speedup vs baseline: 1.5632x; 1.3526x over previous
import functools

import jax
import jax.numpy as jnp
from jax import lax
from jax.experimental import pallas as pl
from jax.experimental.pallas import tpu as pltpu

VOCAB = 1000
BATCH = 4096
SEQ = 20
RB = 32
NB = BATCH // RB


def _onehot_body(idx_ref, tab_ref, out_ref):
    idx_blk = idx_ref[0]
    iot = lax.broadcasted_iota(jnp.int32, (RB, SEQ, VOCAB), 2)
    oh = (iot == idx_blk[:, :, None]).astype(jnp.bfloat16)
    out_ref[...] = lax.dot_general(
        oh, tab_ref[...],
        dimension_numbers=(((2,), (0,)), ((), ())),
        preferred_element_type=jnp.float32)


@jax.jit
def _bigram_logits(idx, table):
    idx3 = idx.reshape(1, BATCH, SEQ)
    table_bf = table.astype(jnp.bfloat16)
    return pl.pallas_call(
        _onehot_body,
        grid=(NB,),
        in_specs=[
            pl.BlockSpec((1, RB, SEQ), lambda i: (0, i, 0)),
            pl.BlockSpec((VOCAB, VOCAB), lambda i: (0, 0)),
        ],
        out_specs=pl.BlockSpec((RB, SEQ, VOCAB), lambda i: (i, 0, 0)),
        out_shape=jax.ShapeDtypeStruct((BATCH, SEQ, VOCAB), jnp.float32),
        compiler_params=pltpu.CompilerParams(
            dimension_semantics=("arbitrary",)),
    )(idx3, table_bf)


def kernel(inputs, table):
    return _bigram_logits(inputs.astype(jnp.int32), table)
